# Initial kernel scaffold; baseline (speedup 1.0000x reference)
import jax
import jax.numpy as jnp
from jax.experimental import pallas as pl


def kernel(x, edge_index, batch, W1, b1, W2, b2, Wfc, bfc):
    raise NotImplementedError("write your pallas kernel here")


# speedup vs baseline: 21.5527x; 21.5527x over previous
import functools

import jax
import jax.numpy as jnp
from jax import lax
from jax.experimental import pallas as pl
from jax.experimental.pallas import tpu as pltpu
from jax.experimental.pallas import tpu_sc as plsc

N = 10000
E = 320000
D = 128
G = 64
NC = 2
NS = 16
NW = NC * NS
CHUNK = 128
K = 79
EP = NW * K * CHUNK
NP = 10240
RPT = NP // NS

_MESH = plsc.VectorSubcoreMesh(
    core_axis_name="c", subcore_axis_name="s", num_cores=NC, num_subcores=NS)


def _zero_vmem_rows(ref, nrows, width):
  z16 = jnp.zeros((16,), jnp.float32)
  per_row = width // 16

  def body(i, _):
    r = i // per_row
    c = (i % per_row) * 16
    ref[r, pl.ds(c, 16)] = z16
    return 0

  lax.fori_loop(0, nrows * per_row, body, 0)


def _sc_deg_body(dst_hbm, part_hbm, dst_v, ones_v, zrow_v, cnt_sh):
  cid = lax.axis_index("c")
  sid = lax.axis_index("s")
  wid = sid * NC + cid

  one16 = jnp.ones((16,), jnp.float32)

  def fill_ones(i, _):
    ones_v[i, pl.ds(0, 16)] = one16
    return 0

  lax.fori_loop(0, CHUNK, fill_ones, 0)
  _zero_vmem_rows(zrow_v, 16, 16)

  def zero_cnt(j, _):
    pltpu.sync_copy(zrow_v, cnt_sh.at[pl.ds(sid * RPT + j * 16, 16)])
    return 0

  lax.fori_loop(0, RPT // 16, zero_cnt, 0)
  pltpu.sync_copy(dst_hbm.at[wid], dst_v)
  plsc.subcore_barrier()

  def chunk(j, _):
    pltpu.sync_copy(ones_v, cnt_sh.at[dst_v.at[j]], add=True)
    return 0

  lax.fori_loop(0, K, chunk, 0)
  plsc.subcore_barrier()
  pltpu.sync_copy(cnt_sh.at[pl.ds(sid * RPT, RPT)],
                  part_hbm.at[cid, pl.ds(sid * RPT, RPT)])


_sc_deg = pl.kernel(
    _sc_deg_body,
    out_type=jax.ShapeDtypeStruct((NC, NP, 16), jnp.float32),
    mesh=_MESH,
    scratch_types=[
        pltpu.VMEM((K, CHUNK), jnp.int32),
        pltpu.VMEM((CHUNK, 16), jnp.float32),
        pltpu.VMEM((16, 16), jnp.float32),
        pltpu.VMEM_SHARED((NP, 16), jnp.float32),
    ],
)


def _sc_agg_body(y_hbm, src_hbm, dst_hbm, part_hbm,
                 src_v, dst_v, buf_v, zrow_v, acc_sh, gsem):
  cid = lax.axis_index("c")
  sid = lax.axis_index("s")
  wid = sid * NC + cid

  _zero_vmem_rows(zrow_v, 16, D)

  def zero_acc(j, _):
    pltpu.sync_copy(zrow_v, acc_sh.at[pl.ds(sid * RPT + j * 16, 16)])
    return 0

  lax.fori_loop(0, RPT // 16, zero_acc, 0)
  pltpu.sync_copy(src_hbm.at[wid], src_v)
  pltpu.sync_copy(dst_hbm.at[wid], dst_v)
  plsc.subcore_barrier()

  def chunk(j, _):
    pltpu.async_copy(y_hbm.at[src_v.at[j]], buf_v, gsem).wait()
    pltpu.sync_copy(buf_v, acc_sh.at[dst_v.at[j]], add=True)
    return 0

  lax.fori_loop(0, K, chunk, 0)
  plsc.subcore_barrier()
  pltpu.sync_copy(acc_sh.at[pl.ds(sid * RPT, RPT)],
                  part_hbm.at[cid, pl.ds(sid * RPT, RPT)])


_sc_agg = pl.kernel(
    _sc_agg_body,
    out_type=jax.ShapeDtypeStruct((NC, NP, D), jnp.float32),
    mesh=_MESH,
    scratch_types=[
        pltpu.VMEM((K, CHUNK), jnp.int32),
        pltpu.VMEM((K, CHUNK), jnp.int32),
        pltpu.VMEM((CHUNK, D), jnp.float32),
        pltpu.VMEM((16, D), jnp.float32),
        pltpu.VMEM_SHARED((NP, D), jnp.float32),
        pltpu.SemaphoreType.DMA,
    ],
)


_HI = lax.Precision.HIGHEST


def _tc1_body(x_ref, w1_ref, c0_ref, c1_ref, y_ref, dinv_ref):
  deg = 1.0 + c0_ref[...] + c1_ref[...]
  dinv = lax.rsqrt(deg)
  z = jnp.dot(x_ref[...], w1_ref[...],
              preferred_element_type=jnp.float32, precision=_HI)
  y_ref[...] = dinv * z
  dinv_ref[...] = dinv


def _tc2_body(a0_ref, a1_ref, y1_ref, dinv_ref, w2_ref, b1_ref, y2_ref):
  dinv = dinv_ref[...]
  h = jnp.maximum(
      dinv * (a0_ref[...] + a1_ref[...] + y1_ref[...]) + b1_ref[...], 0.0)
  z2 = jnp.dot(h, w2_ref[...],
               preferred_element_type=jnp.float32, precision=_HI)
  y2_ref[...] = dinv * z2


def _tc3_body(a0_ref, a1_ref, y2_ref, dinv_ref, b2_ref, batch_ref,
              wfc_ref, bfc_ref, out_ref):
  h = jnp.maximum(
      dinv_ref[...] * (a0_ref[...] + a1_ref[...] + y2_ref[...])
      + b2_ref[...], 0.0)
  gids = lax.broadcasted_iota(jnp.int32, (G, 1), 0)
  onehot = (batch_ref[...] == gids).astype(jnp.float32)
  sums = jnp.dot(onehot, h, preferred_element_type=jnp.float32,
                 precision=_HI)
  cnt = jnp.sum(onehot, axis=1)[:, None]
  pooled = sums / jnp.maximum(cnt, 1.0)
  out_ref[...] = jnp.dot(pooled, wfc_ref[...],
                         preferred_element_type=jnp.float32,
                         precision=_HI) + bfc_ref[...]


def _tc_call(body, out_shapes):
  return pl.pallas_call(body, out_shape=out_shapes)


def kernel(x, edge_index, batch, W1, b1, W2, b2, Wfc, bfc):
  src = edge_index[0].astype(jnp.int32)
  dst = edge_index[1].astype(jnp.int32)
  npad = EP - E
  pad_src = (jnp.arange(npad, dtype=jnp.int32) * 131) % N
  pad_dst = N + jnp.arange(npad, dtype=jnp.int32) % (NP - N)
  srcp = jnp.concatenate([src, pad_src]).reshape(NW, K, CHUNK)
  dstp = jnp.concatenate([dst, pad_dst]).reshape(NW, K, CHUNK)

  cnt_part = _sc_deg(dstp)
  c0 = cnt_part[0, :N, :1]
  c1 = cnt_part[1, :N, :1]

  y1, dinv = _tc_call(_tc1_body, (
      jax.ShapeDtypeStruct((N, D), jnp.float32),
      jax.ShapeDtypeStruct((N, 1), jnp.float32),
  ))(x, W1, c0, c1)

  ap1 = _sc_agg(y1, srcp, dstp)

  y2 = _tc_call(_tc2_body, jax.ShapeDtypeStruct((N, D), jnp.float32))(
      ap1[0, :N], ap1[1, :N], y1, dinv, W2, b1.reshape(1, D))

  ap2 = _sc_agg(y2, srcp, dstp)

  out = _tc_call(_tc3_body, jax.ShapeDtypeStruct((G, 1), jnp.float32))(
      ap2[0, :N], ap2[1, :N], y2, dinv, b2.reshape(1, D),
      batch.astype(jnp.int32).reshape(1, N), Wfc, bfc.reshape(1, 1))

  return out[:, 0]
